# Initial kernel scaffold; baseline (speedup 1.0000x reference)
import jax
import jax.numpy as jnp
from jax.experimental import pallas as pl


def kernel(x, edge_index, edge_weight_params, W1, b1, W2, b2, W3, b3):
    raise NotImplementedError("write your pallas kernel here")


# speedup vs baseline: 1.3390x; 1.3390x over previous
import functools

import jax
import jax.numpy as jnp
from jax.experimental import pallas as pl
from jax.experimental.pallas import tpu as pltpu

_N = 10000
_E = 320000
_D = 128
_H = 128
_C = 16
_INDEX = 123
_NP = 10240
_BR = 512


def _mm_kernel(x_ref, w_ref, b_ref, o_ref, *, relu):
    acc = jnp.dot(x_ref[...], w_ref[...], preferred_element_type=jnp.float32)
    acc = acc + b_ref[...]
    if relu:
        acc = jnp.maximum(acc, 0.0)
    o_ref[...] = acc


def _mm(x, W, b, relu):
    M, K = x.shape[0], W.shape[1]
    return pl.pallas_call(
        functools.partial(_mm_kernel, relu=relu),
        grid=(M // _BR,),
        in_specs=[
            pl.BlockSpec((_BR, x.shape[1]), lambda i: (i, 0)),
            pl.BlockSpec((W.shape[0], K), lambda i: (0, 0)),
            pl.BlockSpec((1, K), lambda i: (0, 0)),
        ],
        out_specs=pl.BlockSpec((_BR, K), lambda i: (i, 0)),
        out_shape=jax.ShapeDtypeStruct((M, K), jnp.float32),
    )(x, W, b.reshape(1, -1))


def kernel(x, edge_index, edge_weight_params, W1, b1, W2, b2, W3, b3):
    src = edge_index[0].astype(jnp.int32)
    dst = edge_index[1].astype(jnp.int32)
    ew = jax.nn.sigmoid(edge_weight_params)

    deg = jax.ops.segment_sum(ew, dst, num_segments=_N) + 1.0
    dinv = jax.lax.rsqrt(deg)
    norm = dinv[src] * ew * dinv[dst]
    dinv2 = dinv * dinv

    xp = jnp.pad(x, ((0, _NP - _N), (0, 0)))

    def conv(inp, W, b, relu):
        msg = inp[src] * norm[:, None]
        agg = jax.ops.segment_sum(msg, dst, num_segments=_NP)
        agg = agg + dinv2[:, None] * inp[:_N] if False else agg.at[:_N].add(dinv2[:, None] * inp[:_N])
        return _mm(agg, W, b, relu)

    h1 = conv(xp[:_N], W1, b1, True)
    h2 = conv(h1[:_N], W2, b2, True)
    aggv = jax.ops.segment_sum(
        jnp.where(dst == _INDEX, norm, 0.0)[:, None] * h2[src], jnp.zeros_like(dst),
        num_segments=1)
    aggv = aggv + dinv2[_INDEX] * h2[_INDEX][None, :]
    z = (aggv @ W3 + b3)[0]
    return jax.nn.log_softmax(z)[()] if z.ndim == 0 else jax.nn.log_softmax(z)
